# baseline (speedup 1.0000x reference)
import jax
import jax.numpy as jnp
from jax import lax
from jax.experimental import pallas as pl
from jax.experimental.pallas import tpu as pltpu
from jax.experimental.pallas import tpu_sc as plsc

VOCAB = 100000
EMBED = 32
BATCH = 1024

NUM_SC = 2
NUM_SUBCORES = 16
NUM_WORKERS = NUM_SC * NUM_SUBCORES
B_PER_W = BATCH // NUM_WORKERS

V_TILE = 1024
NBUF = 4
N_STEPS = 48
LAST_W = VOCAB - (N_STEPS - 1) * V_TILE


def _gather_body(table_hbm, idx_hbm, out_hbm, idx_v, rows_v, sem):
    wid = lax.axis_index("s") * NUM_SC + lax.axis_index("c")
    base = wid * B_PER_W
    pltpu.sync_copy(idx_hbm.at[pl.ds(base, B_PER_W)], idx_v)
    pltpu.async_copy(table_hbm.at[idx_v], rows_v, sem).wait()
    pltpu.sync_copy(rows_v, out_hbm.at[pl.ds(base, B_PER_W)])


_sc_gather = pl.kernel(
    _gather_body,
    mesh=plsc.VectorSubcoreMesh(core_axis_name="c", subcore_axis_name="s"),
    out_type=jax.ShapeDtypeStruct((BATCH, EMBED), jnp.float32),
    scratch_types=[
        pltpu.VMEM((B_PER_W,), jnp.int32),
        pltpu.VMEM((B_PER_W, EMBED), jnp.float32),
        pltpu.SemaphoreType.DMA,
    ],
    compiler_params=pltpu.CompilerParams(use_tc_tiling_on_sc=False),
)


def _proj_body(h_ref, w_ref, b_ref, o_ref):
    o_ref[...] = lax.dot_general(
        w_ref[...], h_ref[...],
        dimension_numbers=(((1,), (1,)), ((), ())),
        preferred_element_type=jnp.float32,
    ) + lax.broadcast_in_dim(b_ref[...], (V_TILE, BATCH), (0,))


def _project_t(h, lin_weight, bias_col):
    return pl.pallas_call(
        _proj_body,
        grid=(pl.cdiv(VOCAB, V_TILE),),
        in_specs=[
            pl.BlockSpec((BATCH, EMBED), lambda j: (0, 0)),
            pl.BlockSpec((V_TILE, EMBED), lambda j: (j, 0)),
            pl.BlockSpec((V_TILE,), lambda j: (j,)),
        ],
        out_specs=pl.BlockSpec((V_TILE, BATCH), lambda j: (j, 0)),
        out_shape=jax.ShapeDtypeStruct((VOCAB, BATCH), jnp.float32),
        compiler_params=pltpu.CompilerParams(
            dimension_semantics=("parallel",)),
    )(h, lin_weight, bias_col)


def kernel(batch, emb_weight, lin_weight, lin_bias):
    idx = batch.astype(jnp.int32)
    h = _sc_gather(emb_weight, idx)
    return _project_t(h, lin_weight, lin_bias).T


# speedup vs baseline: 1.0969x; 1.0969x over previous
import jax
import jax.numpy as jnp
from jax import lax
from jax.experimental import pallas as pl
from jax.experimental.pallas import tpu as pltpu
from jax.experimental.pallas import tpu_sc as plsc

VOCAB = 100000
EMBED = 32
BATCH = 1024

NUM_SC = 2
NUM_SUBCORES = 16
NUM_WORKERS = NUM_SC * NUM_SUBCORES
B_PER_W = BATCH // NUM_WORKERS

V_TILE = 2048
NBUF = 4
N_STEPS = (VOCAB + V_TILE - 1) // V_TILE
LAST_H = VOCAB - (N_STEPS - 1) * V_TILE


def _gather_body(table_hbm, idx_hbm, out_hbm, idx_v, rows_v, sem):
    wid = lax.axis_index("s") * NUM_SC + lax.axis_index("c")
    base = wid * B_PER_W
    pltpu.sync_copy(idx_hbm.at[pl.ds(base, B_PER_W)], idx_v)
    pltpu.async_copy(table_hbm.at[idx_v], rows_v, sem).wait()
    pltpu.sync_copy(rows_v, out_hbm.at[pl.ds(base, B_PER_W)])


_sc_gather = pl.kernel(
    _gather_body,
    mesh=plsc.VectorSubcoreMesh(core_axis_name="c", subcore_axis_name="s"),
    out_type=jax.ShapeDtypeStruct((BATCH, EMBED), jnp.float32),
    scratch_types=[
        pltpu.VMEM((B_PER_W,), jnp.int32),
        pltpu.VMEM((B_PER_W, EMBED), jnp.float32),
        pltpu.SemaphoreType.DMA,
    ],
    compiler_params=pltpu.CompilerParams(use_tc_tiling_on_sc=False),
)


def _row_copy(acc, k, o_hbm, row, rows, sems):
    row = pl.multiple_of(row, V_TILE)
    return pltpu.make_async_copy(
        acc.at[k, pl.ds(0, rows)], o_hbm.at[pl.ds(row, rows)], sems.at[k])


def _proj_body(h_ref, w_ref, b_ref, o_hbm, acc, sems):
    j = pl.program_id(0)

    for k in range(NBUF):
        @pl.when(lax.rem(j, NBUF) == k)
        def _(k=k):
            @pl.when(j >= NBUF)
            def _():
                _row_copy(acc, k, o_hbm, (j - NBUF) * V_TILE, V_TILE,
                          sems).wait()

            acc[k] = lax.dot_general(
                w_ref[...], h_ref[...],
                dimension_numbers=(((1,), (1,)), ((), ())),
                preferred_element_type=jnp.float32,
            ) + lax.broadcast_in_dim(b_ref[...], (V_TILE, BATCH), (0,))

            @pl.when(j != N_STEPS - 1)
            def _():
                _row_copy(acc, k, o_hbm, j * V_TILE, V_TILE, sems).start()

    @pl.when(j == N_STEPS - 1)
    def _():
        kl = (N_STEPS - 1) % NBUF
        _row_copy(acc, kl, o_hbm, (N_STEPS - 1) * V_TILE, LAST_H,
                  sems).start()
        for d in range(NBUF - 1, 0, -1):
            jj = N_STEPS - 1 - d
            _row_copy(acc, jj % NBUF, o_hbm, jj * V_TILE, V_TILE,
                      sems).wait()
        _row_copy(acc, kl, o_hbm, (N_STEPS - 1) * V_TILE, LAST_H,
                  sems).wait()


def _project_t(h, lin_weight, lin_bias):
    return pl.pallas_call(
        _proj_body,
        grid=(N_STEPS,),
        in_specs=[
            pl.BlockSpec((BATCH, EMBED), lambda j: (0, 0)),
            pl.BlockSpec((V_TILE, EMBED), lambda j: (j, 0)),
            pl.BlockSpec((V_TILE,), lambda j: (j,)),
        ],
        out_specs=pl.BlockSpec(memory_space=pl.ANY),
        out_shape=jax.ShapeDtypeStruct((VOCAB, BATCH), jnp.float32),
        scratch_shapes=[
            pltpu.VMEM((NBUF, V_TILE, BATCH), jnp.float32),
            pltpu.SemaphoreType.DMA((NBUF,)),
        ],
        compiler_params=pltpu.CompilerParams(
            dimension_semantics=("arbitrary",)),
    )(h, lin_weight, lin_bias)


def kernel(batch, emb_weight, lin_weight, lin_bias):
    idx = batch.astype(jnp.int32)
    h = _sc_gather(emb_weight, idx)
    return _project_t(h, lin_weight, lin_bias).T


# speedup vs baseline: 1.1140x; 1.0156x over previous
import jax
import jax.numpy as jnp
from jax import lax
from jax.experimental import pallas as pl
from jax.experimental.pallas import tpu as pltpu
from jax.experimental.pallas import tpu_sc as plsc

VOCAB = 100000
EMBED = 32
BATCH = 1024

NUM_SC = 2
NUM_SUBCORES = 16
NUM_WORKERS = NUM_SC * NUM_SUBCORES
B_PER_W = BATCH // NUM_WORKERS

V_TILE = 6144
NBUF = 4
N_STEPS = (VOCAB + V_TILE - 1) // V_TILE
LAST_H = VOCAB - (N_STEPS - 1) * V_TILE


def _gather_body(table_hbm, idx_hbm, out_hbm, idx_v, rows_v, sem):
    wid = lax.axis_index("s") * NUM_SC + lax.axis_index("c")
    base = wid * B_PER_W
    pltpu.sync_copy(idx_hbm.at[pl.ds(base, B_PER_W)], idx_v)
    pltpu.async_copy(table_hbm.at[idx_v], rows_v, sem).wait()
    pltpu.sync_copy(rows_v, out_hbm.at[pl.ds(base, B_PER_W)])


_sc_gather = pl.kernel(
    _gather_body,
    mesh=plsc.VectorSubcoreMesh(core_axis_name="c", subcore_axis_name="s"),
    out_type=jax.ShapeDtypeStruct((BATCH, EMBED), jnp.float32),
    scratch_types=[
        pltpu.VMEM((B_PER_W,), jnp.int32),
        pltpu.VMEM((B_PER_W, EMBED), jnp.float32),
        pltpu.SemaphoreType.DMA,
    ],
    compiler_params=pltpu.CompilerParams(use_tc_tiling_on_sc=False),
)


def _proj_body(h_ref, w_ref, b_ref, o_ref):
    o_ref[...] = lax.dot_general(
        w_ref[...], h_ref[...],
        dimension_numbers=(((1,), (1,)), ((), ())),
        preferred_element_type=jnp.float32,
    ) + lax.broadcast_in_dim(b_ref[...], (V_TILE, BATCH), (0,))


def _project_t(h, lin_weight, lin_bias):
    return pl.pallas_call(
        _proj_body,
        grid=(pl.cdiv(VOCAB, V_TILE),),
        in_specs=[
            pl.BlockSpec((BATCH, EMBED), lambda j: (0, 0)),
            pl.BlockSpec((V_TILE, EMBED), lambda j: (j, 0)),
            pl.BlockSpec((V_TILE,), lambda j: (j,)),
        ],
        out_specs=pl.BlockSpec((V_TILE, BATCH), lambda j: (j, 0)),
        out_shape=jax.ShapeDtypeStruct((VOCAB, BATCH), jnp.float32),
        compiler_params=pltpu.CompilerParams(
            dimension_semantics=("parallel",)),
    )(h, lin_weight, lin_bias)


def kernel(batch, emb_weight, lin_weight, lin_bias):
    idx = batch.astype(jnp.int32)
    h = _sc_gather(emb_weight, idx)
    return _project_t(h, lin_weight, lin_bias).T
